# baseline (speedup 1.0000x reference)
import jax
import jax.numpy as jnp
from jax import lax
from jax.experimental import pallas as pl
from jax.experimental.pallas import tpu as pltpu
from jax.experimental.pallas import tpu_sc as plsc

_M = 1_000_000
_D = 8
_NW = 32
_CW = 3_968
_NCHUNK = 252
_TAIL0 = _NCHUNK * _CW
_TAILW = _M - _TAIL0
_BW = 128
_NSLOT = 8
_NBUF = 4
_LAG = _NBUF // 2
_NUPD = 4


def _copy_body(x_hbm, out_hbm, b0, b1, b2, b3, rsems, wsems):
    wid = lax.axis_index("s") * 2 + lax.axis_index("c")
    bufs = (b0, b1, b2, b3)

    def chunk_id(k):
        return wid + k * _NW

    def valid(k):
        return chunk_id(k) < _NCHUNK

    def rd(k):
        b = k % _NBUF
        col0 = chunk_id(k) * _CW
        return pltpu.make_async_copy(
            x_hbm.at[:, pl.ds(col0, _CW)], bufs[b], rsems.at[b])

    def wr(k):
        b = k % _NBUF
        col0 = chunk_id(k) * _CW
        return pltpu.make_async_copy(
            bufs[b], out_hbm.at[:, pl.ds(col0, _CW)], wsems.at[b])

    for k in range(_LAG):
        pl.when(valid(k))(lambda k=k: rd(k).start())
    for k in range(_NSLOT):
        if k >= _LAG:
            pl.when(valid(k - _LAG))(lambda k=k: wr(k - _LAG).wait())
        if k + _LAG < _NSLOT:
            pl.when(valid(k + _LAG))(lambda k=k: rd(k + _LAG).start())

        def _proc(k=k):
            rd(k).wait()
            wr(k).start()

        pl.when(valid(k))(_proc)
    for k in range(_NSLOT - _LAG, _NSLOT):
        pl.when(valid(k))(lambda k=k: wr(k).wait())


def _sc_copy(xt):
    mesh = plsc.VectorSubcoreMesh(
        core_axis_name="c", subcore_axis_name="s", num_cores=2, num_subcores=16
    )
    return pl.kernel(
        _copy_body,
        out_type=jax.ShapeDtypeStruct((_D, _M), jnp.float32),
        mesh=mesh,
        scratch_types=[
            pltpu.VMEM((_D, _CW), jnp.float32),
            pltpu.VMEM((_D, _CW), jnp.float32),
            pltpu.VMEM((_D, _CW), jnp.float32),
            pltpu.VMEM((_D, _CW), jnp.float32),
            pltpu.SemaphoreType.DMA((_NBUF,)),
            pltpu.SemaphoreType.DMA((_NBUF,)),
        ],
    )(xt)


def _fix_body(idx_ref, updt_ref, xt_ref, tab_ref, out_ref, wbuf, tbuf, sem):
    toff = pl.multiple_of(idx_ref[0] * 0 + _TAIL0, 128)
    pltpu.make_async_copy(xt_ref.at[:, pl.ds(toff, 128)], tbuf, sem).start()
    pltpu.make_async_copy(xt_ref.at[:, pl.ds(toff, 128)], tbuf, sem).wait()
    pltpu.make_async_copy(tbuf, out_ref.at[:, pl.ds(toff, 128)], sem).start()
    pltpu.make_async_copy(tbuf, out_ref.at[:, pl.ds(toff, 128)], sem).wait()
    lane = lax.broadcasted_iota(jnp.int32, (_D, 128), 1)
    for j in range(_NUPD):
        col = idx_ref[j]
        col0 = pl.multiple_of((col // 128) * 128, 128)
        pltpu.make_async_copy(out_ref.at[:, pl.ds(col0, 128)], wbuf, sem).start()
        pltpu.make_async_copy(out_ref.at[:, pl.ds(col0, 128)], wbuf, sem).wait()
        upd = jnp.where(lane == col - col0, updt_ref[:, pl.ds(j, 1)], 0.0)
        wbuf[...] = wbuf[...] + upd
        pltpu.make_async_copy(wbuf, out_ref.at[:, pl.ds(col0, 128)], sem).start()
        pltpu.make_async_copy(wbuf, out_ref.at[:, pl.ds(col0, 128)], sem).wait()


def _tc_fixup(tabt, updt, xt, index):
    grid_spec = pltpu.PrefetchScalarGridSpec(
        num_scalar_prefetch=1,
        grid=(1,),
        in_specs=[
            pl.BlockSpec((_D, _NUPD), lambda i, idx: (0, 0)),
            pl.BlockSpec(memory_space=pl.ANY),
            pl.BlockSpec(memory_space=pl.ANY),
        ],
        out_specs=pl.BlockSpec(memory_space=pl.ANY),
        scratch_shapes=[
            pltpu.VMEM((_D, 128), jnp.float32),
            pltpu.VMEM((_D, 128), jnp.float32),
            pltpu.SemaphoreType.DMA,
        ],
    )
    return pl.pallas_call(
        _fix_body,
        grid_spec=grid_spec,
        out_shape=jax.ShapeDtypeStruct((_D, _M), jnp.float32),
        input_output_aliases={3: 0},
    )(index, updt, xt, tabt)


def kernel(x, update, index):
    xt = x.T
    fixed = _tc_fixup(_sc_copy(xt), update.T, xt, index)
    return fixed.T


# speedup vs baseline: 1.1211x; 1.1211x over previous
import jax
import jax.numpy as jnp
from jax import lax
from jax.experimental import pallas as pl
from jax.experimental.pallas import tpu as pltpu
from jax.experimental.pallas import tpu_sc as plsc

_M = 1_000_000
_D = 8
_NW = 32
_CW = 3_968
_NCHUNK = 252
_TAIL0 = _NCHUNK * _CW
_TAILW = _M - _TAIL0
_BW = 128
_NSLOT = 8
_NBUF = 4
_LAG = _NBUF // 2
_NUPD = 4


def _copy_body(x_hbm, out_hbm, b0, b1, b2, b3, rsems, wsems):
    wid = lax.axis_index("s") * 2 + lax.axis_index("c")
    bufs = (b0, b1, b2, b3)

    def chunk_id(k):
        return wid + k * _NW

    def valid(k):
        return chunk_id(k) < _NCHUNK

    def rd(k):
        b = k % _NBUF
        col0 = chunk_id(k) * _CW
        return pltpu.make_async_copy(
            x_hbm.at[:, pl.ds(col0, _CW)], bufs[b], rsems.at[b])

    def wr(k):
        b = k % _NBUF
        col0 = chunk_id(k) * _CW
        return pltpu.make_async_copy(
            bufs[b], out_hbm.at[:, pl.ds(col0, _CW)], wsems.at[b])

    for k in range(_LAG):
        pl.when(valid(k))(lambda k=k: rd(k).start())
    for k in range(_NSLOT):
        if k >= _LAG:
            pl.when(valid(k - _LAG))(lambda k=k: wr(k - _LAG).wait())
        if k + _LAG < _NSLOT:
            pl.when(valid(k + _LAG))(lambda k=k: rd(k + _LAG).start())

        def _proc(k=k):
            rd(k).wait()
            wr(k).start()

        pl.when(valid(k))(_proc)
    for k in range(_NSLOT - _LAG, _NSLOT):
        pl.when(valid(k))(lambda k=k: wr(k).wait())

    @pl.when(wid == _NW - 1)
    def _tail():
        toff = pl.multiple_of(_TAIL0 + 0 * wid, 128)
        tb = b0.at[:, pl.ds(0, 128)]
        pltpu.make_async_copy(x_hbm.at[:, pl.ds(toff, 128)], tb, rsems.at[0]).start()
        pltpu.make_async_copy(x_hbm.at[:, pl.ds(toff, 128)], tb, rsems.at[0]).wait()
        pltpu.make_async_copy(tb, out_hbm.at[:, pl.ds(toff, 128)], wsems.at[0]).start()
        pltpu.make_async_copy(tb, out_hbm.at[:, pl.ds(toff, 128)], wsems.at[0]).wait()


def _sc_copy(xt):
    mesh = plsc.VectorSubcoreMesh(
        core_axis_name="c", subcore_axis_name="s", num_cores=2, num_subcores=16
    )
    return pl.kernel(
        _copy_body,
        out_type=jax.ShapeDtypeStruct((_D, _M), jnp.float32),
        mesh=mesh,
        scratch_types=[
            pltpu.VMEM((_D, _CW), jnp.float32),
            pltpu.VMEM((_D, _CW), jnp.float32),
            pltpu.VMEM((_D, _CW), jnp.float32),
            pltpu.VMEM((_D, _CW), jnp.float32),
            pltpu.SemaphoreType.DMA((_NBUF,)),
            pltpu.SemaphoreType.DMA((_NBUF,)),
        ],
    )(xt)


def _fix_body(idx_ref, x_blk, updt_ref, out_blk):
    i = pl.program_id(0)
    bid = idx_ref[i] // _BW
    col = bid * _BW + lax.broadcasted_iota(jnp.int32, (_D, _BW), 1)
    acc = x_blk[...]
    for j in range(_NUPD):
        acc = acc + jnp.where(col == idx_ref[j], updt_ref[:, pl.ds(j, 1)], 0.0)
    out_blk[...] = acc


def _block_map(i, idx_ref):
    return (0, idx_ref[i] // _BW)


def _tc_fixup(tabt, updt, xt, index):
    grid_spec = pltpu.PrefetchScalarGridSpec(
        num_scalar_prefetch=1,
        grid=(_NUPD,),
        in_specs=[
            pl.BlockSpec((_D, _BW), _block_map),
            pl.BlockSpec((_D, _NUPD), lambda i, idx: (0, 0)),
        ],
        out_specs=pl.BlockSpec((_D, _BW), _block_map),
    )

    def body(idx_ref, x_blk, updt_ref, tab_ref, out_blk):
        del tab_ref
        _fix_body(idx_ref, x_blk, updt_ref, out_blk)

    grid_spec2 = pltpu.PrefetchScalarGridSpec(
        num_scalar_prefetch=1,
        grid=(_NUPD,),
        in_specs=[
            pl.BlockSpec((_D, _BW), _block_map),
            pl.BlockSpec((_D, _NUPD), lambda i, idx: (0, 0)),
            pl.BlockSpec(memory_space=pl.ANY),
        ],
        out_specs=pl.BlockSpec((_D, _BW), _block_map),
    )
    return pl.pallas_call(
        body,
        grid_spec=grid_spec2,
        out_shape=jax.ShapeDtypeStruct((_D, _M), jnp.float32),
        input_output_aliases={3: 0},
    )(index, xt, updt, tabt)


def kernel(x, update, index):
    xt = x.T
    fixed = _tc_fixup(_sc_copy(xt), update.T, xt, index)
    return fixed.T
